# baseline (speedup 1.0000x reference)
import functools

import jax
import jax.numpy as jnp
import numpy as np
from jax import lax
from jax.experimental import pallas as pl
from jax.experimental.pallas import tpu as pltpu
from jax.experimental.pallas import tpu_sc as plsc

BATCH = 16384
SEQ_LEN = 50
DIM = 64

_info = plsc.get_sparse_core_info()
_NC, _NS, _L = _info.num_cores, _info.num_subcores, _info.num_lanes
_NW = _NC * _NS

_SAMPLES_PER_ROW = 2
_IDX_PER_ROW = _SAMPLES_PER_ROW * SEQ_LEN
_ROWS_PER_BLOCK = 8
_SAMPLES_PER_BLOCK = _ROWS_PER_BLOCK * _SAMPLES_PER_ROW
_SAMPLES_PER_WORKER = BATCH // _NW
_BLOCKS_PER_WORKER = _SAMPLES_PER_WORKER // _SAMPLES_PER_BLOCK
_X_ROWS_PER_WORKER = _SAMPLES_PER_WORKER // _SAMPLES_PER_ROW

_PERM = np.concatenate([np.arange(0, 32, 2), np.arange(1, 32, 2),
                        np.arange(32, 64, 2), np.arange(33, 64, 2)])
_INV_PERM = np.argsort(_PERM)


def _bow_body(x_hbm, table_hbm, bias_hbm, out_hbm,
              idx_v, rows_v, out_v, bias_v, sem0, sem1, osem0, osem1):
    wid = lax.axis_index("s") * _NC + lax.axis_index("c")
    sems = (sem0, sem1)
    osems = (osem0, osem1)

    pltpu.sync_copy(bias_hbm, bias_v)

    def fire(slot, b):
        row_base = wid * _X_ROWS_PER_WORKER + b * _ROWS_PER_BLOCK
        pltpu.sync_copy(x_hbm.at[pl.ds(row_base, _ROWS_PER_BLOCK), :],
                        idx_v.at[slot])
        for j in range(_ROWS_PER_BLOCK):
            pltpu.async_copy(table_hbm.at[idx_v.at[slot, j]],
                             rows_v.at[slot, j], sems[slot])

    def drain(slot):
        for j in range(_ROWS_PER_BLOCK):
            pltpu.make_async_copy(table_hbm.at[idx_v.at[slot, j]],
                                  rows_v.at[slot, j], sems[slot]).wait()

    def compute(slot, b):
        sample_base = wid * _SAMPLES_PER_WORKER + b * _SAMPLES_PER_BLOCK

        def sample_body(s, _):
            j = s // _SAMPLES_PER_ROW
            off = (s % _SAMPLES_PER_ROW) * SEQ_LEN
            accs = [bias_v[pl.ds(g * _L, _L)] for g in range(4)]
            for r in range(SEQ_LEN):
                lo = rows_v[slot, j, off + r, pl.ds(0, 32)]
                hi = rows_v[slot, j, off + r, pl.ds(32, 32)]
                e0, o0 = plsc.unpack(lo, format=plsc.PackFormat.INTERLEAVED)
                e1, o1 = plsc.unpack(hi, format=plsc.PackFormat.INTERLEAVED)
                accs[0] = accs[0] + e0
                accs[1] = accs[1] + o0
                accs[2] = accs[2] + e1
                accs[3] = accs[3] + o1
            for g in range(4):
                out_v[slot, s, pl.ds(g * _L, _L)] = accs[g]
            return 0

        lax.fori_loop(0, _SAMPLES_PER_BLOCK, sample_body, 0)
        pltpu.async_copy(out_v.at[slot],
                         out_hbm.at[pl.ds(sample_base, _SAMPLES_PER_BLOCK), :],
                         osems[slot])

    def drain_out(slot, b):
        sample_base = wid * _SAMPLES_PER_WORKER + b * _SAMPLES_PER_BLOCK
        pltpu.make_async_copy(
            out_v.at[slot],
            out_hbm.at[pl.ds(sample_base, _SAMPLES_PER_BLOCK), :],
            osems[slot]).wait()

    fire(0, 0)

    def pair_body(i, _):
        for phase in range(2):
            b = 2 * i + phase
            cur, nxt = phase, 1 - phase

            @pl.when(b + 1 < _BLOCKS_PER_WORKER)
            def _():
                fire(nxt, b + 1)

            drain(cur)

            @pl.when(b >= 2)
            def _():
                drain_out(cur, b - 2)

            compute(cur, b)
        return 0

    lax.fori_loop(0, _BLOCKS_PER_WORKER // 2, pair_body, 0)
    drain_out(0, _BLOCKS_PER_WORKER - 2)
    drain_out(1, _BLOCKS_PER_WORKER - 1)


@jax.jit
def _bow_call(x2, tbf, bias_perm):
    mesh = plsc.VectorSubcoreMesh(core_axis_name="c", subcore_axis_name="s")
    f = functools.partial(
        pl.kernel,
        mesh=mesh,
        out_type=jax.ShapeDtypeStruct((BATCH, DIM), jnp.float32),
        scratch_types=[
            pltpu.VMEM((2, _ROWS_PER_BLOCK, _IDX_PER_ROW), jnp.int32),
            pltpu.VMEM((2, _ROWS_PER_BLOCK, _IDX_PER_ROW, DIM), jnp.bfloat16),
            pltpu.VMEM((2, _SAMPLES_PER_BLOCK, DIM), jnp.float32),
            pltpu.VMEM((DIM,), jnp.float32),
            pltpu.SemaphoreType.DMA,
            pltpu.SemaphoreType.DMA,
            pltpu.SemaphoreType.DMA,
            pltpu.SemaphoreType.DMA,
        ],
        compiler_params=pltpu.CompilerParams(use_tc_tiling_on_sc=False,
                                             needs_layout_passes=False),
    )(_bow_body)
    return f(x2, tbf, bias_perm)


def kernel(x, table, bias):
    x2 = x.reshape(BATCH // _SAMPLES_PER_ROW, _IDX_PER_ROW).astype(jnp.int32)
    tbf = table.astype(jnp.bfloat16)
    bias_perm = bias[_PERM]
    logits = _bow_call(x2, tbf, bias_perm)
    logits = logits[:, _INV_PERM]
    return (logits[:, :16], logits[:, 16:32], logits[:, 32:])
